# baseline (speedup 1.0000x reference)
import functools

import jax
import jax.numpy as jnp
from jax.experimental import pallas as pl

BATCH = 1024
INPUT_DIM = 256
NUM_PROTOTYPES = 512
NUM_CLASSES = 8
PER_CLASS = NUM_PROTOTYPES // NUM_CLASSES
LAMBDA_VAL = 1.0


def _glmvq_kernel(x_ref, y_ref, p_ref, omega_ref, out_ref):
    x = x_ref[...]
    cols = []
    omega_sq = jnp.float32(0.0)
    for c in range(NUM_CLASSES):
        om = omega_ref[c]
        om32 = om.astype(jnp.float32)
        omega_sq = omega_sq + jnp.sum(om32 * om32)
        tx = jax.lax.dot_general(
            x, om, (((1,), (1,)), ((), ())),
            preferred_element_type=jnp.float32)
        tp = jax.lax.dot_general(
            p_ref[c], om, (((1,), (1,)), ((), ())),
            preferred_element_type=jnp.float32)
        txb = tx.astype(jnp.bfloat16)
        tpb = tp.astype(jnp.bfloat16)
        norm_tx = jnp.sum(tx * tx, axis=1, keepdims=True)
        norm_tp = jnp.sum(tp * tp, axis=1)
        cross = jax.lax.dot_general(
            txb, tpb, (((1,), (1,)), ((), ())),
            preferred_element_type=jnp.float32)
        dist = norm_tx + norm_tp[None, :] - 2.0 * cross
        cols.append(jnp.min(dist, axis=1, keepdims=True))
    mind = jnp.concatenate(cols, axis=1)
    y = y_ref[...]
    same = jax.lax.broadcasted_iota(jnp.int32, (BATCH, NUM_CLASSES), 1) == y
    inf = jnp.float32(jnp.inf)
    pos = jnp.min(jnp.where(same, mind, inf), axis=1)
    neg = jnp.min(jnp.where(same, inf, mind), axis=1)
    mu = (pos - neg) / (pos + neg)
    loss = jnp.mean(1.0 / (1.0 + jnp.exp(-LAMBDA_VAL * mu)))
    out_ref[...] = (loss + 0.01 * jnp.sqrt(omega_sq)).reshape(1, 1)


@functools.partial(jax.jit, static_argnames=())
def kernel(x, y, prototypes, omega):
    protos_r = prototypes.reshape(PER_CLASS, NUM_CLASSES, INPUT_DIM)
    protos_r = protos_r.transpose(1, 0, 2)
    y2 = y.reshape(BATCH, 1)
    out = pl.pallas_call(
        _glmvq_kernel,
        out_shape=jax.ShapeDtypeStruct((1, 1), jnp.float32),
    )(x.astype(jnp.bfloat16), y2, protos_r.astype(jnp.bfloat16),
      omega.astype(jnp.bfloat16))
    return out[0, 0]


# speedup vs baseline: 1.1756x; 1.1756x over previous
import functools

import jax
import jax.numpy as jnp
from jax.experimental import pallas as pl

BATCH = 1024
INPUT_DIM = 256
NUM_PROTOTYPES = 512
NUM_CLASSES = 8
PER_CLASS = NUM_PROTOTYPES // NUM_CLASSES
LAMBDA_VAL = 1.0


def _glmvq_kernel(x_ref, y_ref, p_ref, omega_ref, out_ref):
    x = x_ref[...]
    cols = []
    omega_sq = jnp.float32(0.0)
    for c in range(NUM_CLASSES):
        om = omega_ref[c]
        omega_sq = omega_sq + jnp.sum(om * om)
        tx = jax.lax.dot_general(
            x, om, (((1,), (1,)), ((), ())),
            preferred_element_type=jnp.float32)
        tp = jax.lax.dot_general(
            p_ref[c], om, (((1,), (1,)), ((), ())),
            preferred_element_type=jnp.float32)
        norm_tx = jnp.sum(tx * tx, axis=1, keepdims=True)
        norm_tp = jnp.sum(tp * tp, axis=1)
        cross = jax.lax.dot_general(
            tx, tp, (((1,), (1,)), ((), ())),
            preferred_element_type=jnp.float32)
        dist = norm_tx + norm_tp[None, :] - 2.0 * cross
        cols.append(jnp.min(dist, axis=1, keepdims=True))
    mind = jnp.concatenate(cols, axis=1)
    y = y_ref[...]
    same = jax.lax.broadcasted_iota(jnp.int32, (BATCH, NUM_CLASSES), 1) == y
    inf = jnp.float32(jnp.inf)
    pos = jnp.min(jnp.where(same, mind, inf), axis=1)
    neg = jnp.min(jnp.where(same, inf, mind), axis=1)
    mu = (pos - neg) / (pos + neg)
    loss = jnp.mean(1.0 / (1.0 + jnp.exp(-LAMBDA_VAL * mu)))
    out_ref[...] = (loss + 0.01 * jnp.sqrt(omega_sq)).reshape(1, 1)


@functools.partial(jax.jit, static_argnames=())
def kernel(x, y, prototypes, omega):
    protos_r = prototypes.reshape(PER_CLASS, NUM_CLASSES, INPUT_DIM)
    protos_r = protos_r.transpose(1, 0, 2)
    y2 = y.reshape(BATCH, 1)
    out = pl.pallas_call(
        _glmvq_kernel,
        out_shape=jax.ShapeDtypeStruct((1, 1), jnp.float32),
    )(x, y2, protos_r, omega)
    return out[0, 0]


# speedup vs baseline: 1.4351x; 1.2207x over previous
import functools

import jax
import jax.numpy as jnp
from jax.experimental import pallas as pl

BATCH = 1024
INPUT_DIM = 256
NUM_PROTOTYPES = 512
NUM_CLASSES = 8
PER_CLASS = NUM_PROTOTYPES // NUM_CLASSES
LAMBDA_VAL = 1.0


def _glmvq_kernel(x_ref, y_ref, p_ref, omega_ref, out_ref):
    x = x_ref[...]
    cols = []
    omega_sq = jnp.float32(0.0)
    for c in range(NUM_CLASSES):
        om = omega_ref[c]
        omega_sq = omega_sq + jnp.sum(om * om)
        tx = jax.lax.dot_general(
            x, om, (((1,), (1,)), ((), ())),
            preferred_element_type=jnp.float32)
        tp = jax.lax.dot_general(
            p_ref[:, c, :], om, (((1,), (1,)), ((), ())),
            preferred_element_type=jnp.float32)
        norm_tx = jnp.sum(tx * tx, axis=1, keepdims=True)
        norm_tp = jnp.sum(tp * tp, axis=1)
        cross = jax.lax.dot_general(
            tx, tp, (((1,), (1,)), ((), ())),
            preferred_element_type=jnp.float32)
        dist = norm_tx + norm_tp[None, :] - 2.0 * cross
        cols.append(jnp.min(dist, axis=1, keepdims=True))
    mind = jnp.concatenate(cols, axis=1)
    y = y_ref[...]
    same = jax.lax.broadcasted_iota(jnp.int32, (BATCH, NUM_CLASSES), 1) == y
    inf = jnp.float32(jnp.inf)
    pos = jnp.min(jnp.where(same, mind, inf), axis=1)
    neg = jnp.min(jnp.where(same, inf, mind), axis=1)
    mu = (pos - neg) / (pos + neg)
    loss = jnp.mean(1.0 / (1.0 + jnp.exp(-LAMBDA_VAL * mu)))
    out_ref[...] = (loss + 0.01 * jnp.sqrt(omega_sq)).reshape(1, 1)


@functools.partial(jax.jit, static_argnames=())
def kernel(x, y, prototypes, omega):
    protos_r = prototypes.reshape(PER_CLASS, NUM_CLASSES, INPUT_DIM)
    y2 = y.reshape(BATCH, 1)
    out = pl.pallas_call(
        _glmvq_kernel,
        out_shape=jax.ShapeDtypeStruct((1, 1), jnp.float32),
    )(x, y2, protos_r, omega)
    return out[0, 0]
